# baseline (speedup 1.0000x reference)
import jax
import jax.numpy as jnp
from jax import lax
from jax.experimental import pallas as pl
from jax.experimental.pallas import tpu as pltpu
from jax.experimental.pallas import tpu_sc as plsc

N = 10000
NPAD = 10240
D = 128
E = 320000
NC, NS, LANES = 2, 16, 16
CH = 128
NCROW0 = 160
EPAD = NS * NCROW0 * CH
NSEG = 4
SEGR = NCROW0 // NSEG
SEGP = SEGR // 2
RPT = NPAD // NS
BLK = 1024
GRID = NPAD // BLK


def _tc1_body(x_ref, w_ref, asr_ref, adr_ref, h_ref, as_ref, ad_ref, mx_ref):
    i = pl.program_id(0)
    h = jnp.dot(x_ref[...], w_ref[...], preferred_element_type=jnp.float32)
    h_ref[...] = h
    a_s = jnp.sum(h * asr_ref[...], axis=1, keepdims=True)
    a_d = jnp.sum(h * adr_ref[...], axis=1, keepdims=True)
    as_ref[...] = a_s
    ad_ref[...] = a_d
    cur = jnp.concatenate(
        [jnp.full((1, D), jnp.max(a_s)), jnp.full((1, D), jnp.max(a_d))], axis=0)

    @pl.when(i == 0)
    def _():
        mx_ref[...] = cur

    @pl.when(i != 0)
    def _():
        mx_ref[...] = jnp.maximum(mx_ref[...], cur)


def _tc1(x_pad, gat_W, att_src_row, att_dst_row):
    return pl.pallas_call(
        _tc1_body,
        grid=(GRID,),
        in_specs=[
            pl.BlockSpec((BLK, D), lambda i: (i, 0)),
            pl.BlockSpec((D, D), lambda i: (0, 0)),
            pl.BlockSpec((1, D), lambda i: (0, 0)),
            pl.BlockSpec((1, D), lambda i: (0, 0)),
        ],
        out_specs=[
            pl.BlockSpec((BLK, D), lambda i: (i, 0)),
            pl.BlockSpec((BLK, 1), lambda i: (i, 0)),
            pl.BlockSpec((BLK, 1), lambda i: (i, 0)),
            pl.BlockSpec((2, D), lambda i: (0, 0)),
        ],
        out_shape=[
            jax.ShapeDtypeStruct((NPAD, D), jnp.float32),
            jax.ShapeDtypeStruct((NPAD, 1), jnp.float32),
            jax.ShapeDtypeStruct((NPAD, 1), jnp.float32),
            jax.ShapeDtypeStruct((2, D), jnp.float32),
        ],
    )(x_pad, gat_W, att_src_row, att_dst_row)


def _tc2_body(h_ref, as_ref, ad_ref, mx_ref, un_ref, dc_ref,
              bias_ref, wroot_ref, x1_ref, root_ref):
    i = pl.program_id(0)
    shift = mx_ref[0, 0] + mx_ref[1, 0]
    z = as_ref[...] + ad_ref[...]
    ea = jnp.exp(jnp.maximum(z, 0.2 * z) - shift)
    den = dc_ref[...] + ea
    un = un_ref[...] + ea * h_ref[...]
    x1 = jnp.maximum(un / den + bias_ref[...], 0.0)
    rowid = lax.broadcasted_iota(jnp.int32, (BLK, 1), 0) + i * BLK
    x1 = jnp.where(rowid < N, x1, 0.0)
    x1_ref[...] = x1
    root_ref[...] = jnp.dot(x1, wroot_ref[...], preferred_element_type=jnp.float32)


def _tc2(h, as_col, ad_col, mx, un, dcol, gat_bias_row, gc_Wroot):
    return pl.pallas_call(
        _tc2_body,
        grid=(GRID,),
        in_specs=[
            pl.BlockSpec((BLK, D), lambda i: (i, 0)),
            pl.BlockSpec((BLK, 1), lambda i: (i, 0)),
            pl.BlockSpec((BLK, 1), lambda i: (i, 0)),
            pl.BlockSpec((2, D), lambda i: (0, 0)),
            pl.BlockSpec((BLK, D), lambda i: (i, 0)),
            pl.BlockSpec((BLK, 1), lambda i: (i, 0)),
            pl.BlockSpec((1, D), lambda i: (0, 0)),
            pl.BlockSpec((D, D), lambda i: (0, 0)),
        ],
        out_specs=[
            pl.BlockSpec((BLK, D), lambda i: (i, 0)),
            pl.BlockSpec((BLK, D), lambda i: (i, 0)),
        ],
        out_shape=[
            jax.ShapeDtypeStruct((NPAD, D), jnp.float32),
            jax.ShapeDtypeStruct((NPAD, D), jnp.float32),
        ],
    )(h, as_col, ad_col, mx, un, dcol, gat_bias_row, gc_Wroot)


def _tc3_body(ag_ref, root_ref, wrel_ref, gcb_ref, w1_ref, b1_ref,
              w2_ref, b2_ref, y_ref):
    x2 = jnp.dot(ag_ref[...], wrel_ref[...], preferred_element_type=jnp.float32)
    x2 = jnp.maximum(x2 + root_ref[...] + gcb_ref[...], 0.0)
    x3 = jnp.maximum(
        jnp.dot(x2, w1_ref[...], preferred_element_type=jnp.float32) + b1_ref[...],
        0.0)
    y_ref[...] = jnp.dot(x3, w2_ref[...], preferred_element_type=jnp.float32) + b2_ref[...]


def _tc3(ag, root, gc_Wrel, gcb_row, lin1_W, b1_row, lin2_W, b2_row):
    return pl.pallas_call(
        _tc3_body,
        grid=(GRID,),
        in_specs=[
            pl.BlockSpec((BLK, D), lambda i: (i, 0)),
            pl.BlockSpec((BLK, D), lambda i: (i, 0)),
            pl.BlockSpec((D, D), lambda i: (0, 0)),
            pl.BlockSpec((1, D), lambda i: (0, 0)),
            pl.BlockSpec((D, D), lambda i: (0, 0)),
            pl.BlockSpec((1, D), lambda i: (0, 0)),
            pl.BlockSpec((D, D), lambda i: (0, 0)),
            pl.BlockSpec((1, D), lambda i: (0, 0)),
        ],
        out_specs=pl.BlockSpec((BLK, D), lambda i: (i, 0)),
        out_shape=jax.ShapeDtypeStruct((NPAD, D), jnp.float32),
    )(ag, root, gc_Wrel, gcb_row, lin1_W, b1_row, lin2_W, b2_row)


def _sc_mesh():
    return plsc.VectorSubcoreMesh(core_axis_name="c", subcore_axis_name="s",
                                  num_cores=NC, num_subcores=NS)


def _zero_rows(rows):
    def _zr(j, _):
        for cc in range(8):
            rows[j, pl.ds(cc * LANES, LANES)] = jnp.zeros((LANES,), jnp.float32)
        return 0
    lax.fori_loop(0, CH, _zr, 0)


def _ecompute(asb, adb, bv, grow, e_buf):
    for j in range(CH // LANES):
        asv = asb[pl.ds(j * LANES, LANES)]
        adv = adb[pl.ds(j * LANES, LANES)]
        z = asv + adv
        z = jnp.maximum(z, 0.2 * z) - bv
        e = jnp.exp(z)
        pos = lax.iota(jnp.int32, LANES) + (grow * CH + j * LANES)
        e = jnp.where(pos < E, e, 0.0)
        e_buf[pl.ds(j * LANES, LANES)] = e


def _scale_rows(rows, e_buf):
    def scale(g, _):
        ev = e_buf[pl.ds(g * LANES, LANES)]
        for t in range(LANES):
            es = ev[t]
            j = g * LANES + t
            for cc in range(8):
                rows[j, pl.ds(cc * LANES, LANES)] = rows[j, pl.ds(cc * LANES, LANES)] * es
        return 0
    lax.fori_loop(0, CH // LANES, scale, 0)


def _sc_gat_body(src_hbm, dst_hbm, asrc_hbm, adst_hbm, h_hbm, bv_hbm,
                 un_out, den_out,
                 bv_v, idx_s2, idx_d2, as0, as1, ad0, ad1, eb0, eb1,
                 rows0, rows1, zbuf, un_sp, den_sp,
                 g0, g1, s0, s1, t0, t1, u0, u1):
    cid = lax.axis_index("c")
    sid = lax.axis_index("s")

    @pl.when(cid == 0)
    def _():
        rowstart = sid * NCROW0

        _zero_rows(rows0)

        def _zz(j, _):
            zbuf[pl.ds(j * LANES, LANES)] = jnp.zeros((LANES,), jnp.float32)
            return 0
        lax.fori_loop(0, RPT // LANES, _zz, 0)

        pltpu.sync_copy(zbuf, den_sp.at[pl.ds(RPT * sid, RPT)])
        for k in range(RPT // CH):
            pltpu.sync_copy(rows0, un_sp.at[pl.ds(RPT * sid + CH * k, CH)])
        plsc.subcore_barrier()

        pltpu.sync_copy(bv_hbm, bv_v)
        bv = bv_v[...]

        for seg in range(NSEG):
            segrow = rowstart + seg * SEGR
            pltpu.sync_copy(src_hbm.at[pl.ds(segrow, SEGR)], idx_s2)
            pltpu.sync_copy(dst_hbm.at[pl.ds(segrow, SEGR)], idx_d2)

            pltpu.async_copy(asrc_hbm.at[idx_s2.at[0]], as0, t0)
            pltpu.async_copy(adst_hbm.at[idx_d2.at[0]], ad0, u0)
            pltpu.async_copy(asrc_hbm.at[idx_s2.at[1]], as1, t1)
            pltpu.async_copy(adst_hbm.at[idx_d2.at[1]], ad1, u1)
            pltpu.async_copy(h_hbm.at[idx_s2.at[0]], rows0, g0)
            pltpu.async_copy(h_hbm.at[idx_s2.at[1]], rows1, g1)

            def pair(k, _):
                l0 = k * 2
                l1 = l0 + 1
                pltpu.make_async_copy(asrc_hbm.at[idx_s2.at[0]], as0, t0).wait()
                pltpu.make_async_copy(adst_hbm.at[idx_d2.at[0]], ad0, u0).wait()
                _ecompute(as0, ad0, bv, segrow + l0, eb0)
                pltpu.sync_copy(eb0, den_sp.at[idx_d2.at[l0]], add=True)

                @pl.when(k > 0)
                def _():
                    pltpu.make_async_copy(rows1, un_sp.at[idx_d2.at[0]], s1).wait()
                    pltpu.async_copy(h_hbm.at[idx_s2.at[l1]], rows1, g1)

                pltpu.make_async_copy(h_hbm.at[idx_s2.at[0]], rows0, g0).wait()
                _scale_rows(rows0, eb0)

                @pl.when(k < SEGP - 1)
                def _():
                    pltpu.async_copy(asrc_hbm.at[idx_s2.at[l0 + 2]], as0, t0)
                    pltpu.async_copy(adst_hbm.at[idx_d2.at[l0 + 2]], ad0, u0)

                pltpu.async_copy(rows0, un_sp.at[idx_d2.at[l0]], s0, add=True)

                pltpu.make_async_copy(asrc_hbm.at[idx_s2.at[0]], as1, t1).wait()
                pltpu.make_async_copy(adst_hbm.at[idx_d2.at[0]], ad1, u1).wait()
                _ecompute(as1, ad1, bv, segrow + l1, eb1)
                pltpu.sync_copy(eb1, den_sp.at[idx_d2.at[l1]], add=True)
                pltpu.make_async_copy(h_hbm.at[idx_s2.at[0]], rows1, g1).wait()
                _scale_rows(rows1, eb1)
                pltpu.make_async_copy(rows0, un_sp.at[idx_d2.at[0]], s0).wait()

                @pl.when(k < SEGP - 1)
                def _():
                    pltpu.async_copy(h_hbm.at[idx_s2.at[l0 + 2]], rows0, g0)
                    pltpu.async_copy(asrc_hbm.at[idx_s2.at[l1 + 2]], as1, t1)
                    pltpu.async_copy(adst_hbm.at[idx_d2.at[l1 + 2]], ad1, u1)

                pltpu.async_copy(rows1, un_sp.at[idx_d2.at[l1]], s1, add=True)
                return 0

            lax.fori_loop(0, SEGP, pair, 0)
            pltpu.make_async_copy(rows1, un_sp.at[idx_d2.at[0]], s1).wait()

        plsc.subcore_barrier()

        pltpu.sync_copy(un_sp.at[pl.ds(RPT * sid, RPT)],
                        un_out.at[pl.ds(RPT * sid, RPT)])
        pltpu.sync_copy(den_sp.at[pl.ds(RPT * sid, RPT)], den_out.at[sid])


def _sc_gat(srcp2, dstp2, asrc, adst, h, bvec):
    return pl.kernel(
        _sc_gat_body,
        out_type=(
            jax.ShapeDtypeStruct((NPAD, D), jnp.float32),
            jax.ShapeDtypeStruct((NS, RPT), jnp.float32),
        ),
        mesh=_sc_mesh(),
        compiler_params=pltpu.CompilerParams(needs_layout_passes=False),
        scratch_types=[
            pltpu.VMEM((LANES,), jnp.float32),
            pltpu.VMEM((SEGR, CH), jnp.int32),
            pltpu.VMEM((SEGR, CH), jnp.int32),
            pltpu.VMEM((CH,), jnp.float32),
            pltpu.VMEM((CH,), jnp.float32),
            pltpu.VMEM((CH,), jnp.float32),
            pltpu.VMEM((CH,), jnp.float32),
            pltpu.VMEM((CH,), jnp.float32),
            pltpu.VMEM((CH,), jnp.float32),
            pltpu.VMEM((CH, D), jnp.float32),
            pltpu.VMEM((CH, D), jnp.float32),
            pltpu.VMEM((RPT,), jnp.float32),
            pltpu.VMEM_SHARED((NPAD, D), jnp.float32),
            pltpu.VMEM_SHARED((NPAD,), jnp.float32),
            pltpu.SemaphoreType.DMA,
            pltpu.SemaphoreType.DMA,
            pltpu.SemaphoreType.DMA,
            pltpu.SemaphoreType.DMA,
            pltpu.SemaphoreType.DMA,
            pltpu.SemaphoreType.DMA,
            pltpu.SemaphoreType.DMA,
            pltpu.SemaphoreType.DMA,
        ],
    )(srcp2, dstp2, asrc, adst, h, bvec)


def _sc_agg_body(src_hbm, dst_hbm, x1_hbm, ag_out,
                 idx_s2, idx_d2, rows0, rows1, agg_sp, g0, g1, s0, s1):
    cid = lax.axis_index("c")
    sid = lax.axis_index("s")

    @pl.when(cid == 0)
    def _():
        rowstart = sid * NCROW0

        _zero_rows(rows0)
        for k in range(RPT // CH):
            pltpu.sync_copy(rows0, agg_sp.at[pl.ds(RPT * sid + CH * k, CH)])
        plsc.subcore_barrier()

        for seg in range(NSEG):
            segrow = rowstart + seg * SEGR
            pltpu.sync_copy(src_hbm.at[pl.ds(segrow, SEGR)], idx_s2)
            pltpu.sync_copy(dst_hbm.at[pl.ds(segrow, SEGR)], idx_d2)

            pltpu.async_copy(x1_hbm.at[idx_s2.at[0]], rows0, g0)
            pltpu.async_copy(x1_hbm.at[idx_s2.at[1]], rows1, g1)

            def pair(k, _):
                l0 = k * 2
                l1 = l0 + 1

                @pl.when(k > 0)
                def _():
                    pltpu.make_async_copy(rows1, agg_sp.at[idx_d2.at[0]], s1).wait()
                    pltpu.async_copy(x1_hbm.at[idx_s2.at[l1]], rows1, g1)

                pltpu.make_async_copy(x1_hbm.at[idx_s2.at[0]], rows0, g0).wait()
                pltpu.async_copy(rows0, agg_sp.at[idx_d2.at[l0]], s0, add=True)

                pltpu.make_async_copy(x1_hbm.at[idx_s2.at[0]], rows1, g1).wait()
                pltpu.make_async_copy(rows0, agg_sp.at[idx_d2.at[0]], s0).wait()

                @pl.when(k < SEGP - 1)
                def _():
                    pltpu.async_copy(x1_hbm.at[idx_s2.at[l0 + 2]], rows0, g0)

                pltpu.async_copy(rows1, agg_sp.at[idx_d2.at[l1]], s1, add=True)
                return 0

            lax.fori_loop(0, SEGP, pair, 0)
            pltpu.make_async_copy(rows1, agg_sp.at[idx_d2.at[0]], s1).wait()

        plsc.subcore_barrier()
        pltpu.sync_copy(agg_sp.at[pl.ds(RPT * sid, RPT)],
                        ag_out.at[pl.ds(RPT * sid, RPT)])


def _sc_agg(srcp2, dstp2, x1):
    return pl.kernel(
        _sc_agg_body,
        out_type=jax.ShapeDtypeStruct((NPAD, D), jnp.float32),
        mesh=_sc_mesh(),
        compiler_params=pltpu.CompilerParams(needs_layout_passes=False),
        scratch_types=[
            pltpu.VMEM((SEGR, CH), jnp.int32),
            pltpu.VMEM((SEGR, CH), jnp.int32),
            pltpu.VMEM((CH, D), jnp.float32),
            pltpu.VMEM((CH, D), jnp.float32),
            pltpu.VMEM_SHARED((NPAD, D), jnp.float32),
            pltpu.SemaphoreType.DMA,
            pltpu.SemaphoreType.DMA,
            pltpu.SemaphoreType.DMA,
            pltpu.SemaphoreType.DMA,
        ],
    )(srcp2, dstp2, x1)


@jax.jit
def kernel(x, edge_index, edge_attr, batch, gat_W, gat_att_src, gat_att_dst,
           gat_bias, gc_Wrel, gc_Wroot, gc_bias, lin1_W, lin1_b, lin2_W,
           lin2_b):
    x_pad = jnp.zeros((NPAD, D), jnp.float32).at[:N].set(x)
    srcp = jnp.concatenate([
        edge_index[0].astype(jnp.int32),
        jnp.full((EPAD - E,), N, jnp.int32),
    ]).reshape(EPAD // CH, CH)
    dstp = jnp.concatenate([
        edge_index[1].astype(jnp.int32),
        jnp.zeros((EPAD - E,), jnp.int32),
    ]).reshape(EPAD // CH, CH)

    h, as_col, ad_col, mx = _tc1(x_pad, gat_W,
                                 gat_att_src.reshape(1, D),
                                 gat_att_dst.reshape(1, D))
    bvec = jnp.full((LANES,), mx[0, 0] + mx[1, 0], jnp.float32)

    un, den = _sc_gat(srcp, dstp, as_col.reshape(NPAD), ad_col.reshape(NPAD),
                      h, bvec)
    dcol = den.reshape(NPAD, 1)

    x1, root = _tc2(h, as_col, ad_col, mx, un, dcol,
                    gat_bias.reshape(1, D), gc_Wroot)

    ag = _sc_agg(srcp, dstp, x1)

    y = _tc3(ag, root, gc_Wrel, gc_bias.reshape(1, D),
             lin1_W, lin1_b.reshape(1, D), lin2_W, lin2_b.reshape(1, D))
    return y[:N]


# speedup vs baseline: 1.0491x; 1.0491x over previous
import jax
import jax.numpy as jnp
from jax import lax
from jax.experimental import pallas as pl
from jax.experimental.pallas import tpu as pltpu
from jax.experimental.pallas import tpu_sc as plsc

N = 10000
NPAD = 10240
D = 128
E = 320000
NC, NS, LANES = 2, 16, 16
CH = 128
NCROW0 = 160
EPAD = NS * NCROW0 * CH
NSEG = 4
SEGR = NCROW0 // NSEG
SEGP = SEGR // 2
RPT = NPAD // NS
BLK = 1024
GRID = NPAD // BLK


def _tc1_body(x_ref, w_ref, asr_ref, adr_ref, h_ref, as_ref, ad_ref, mx_ref):
    i = pl.program_id(0)
    h = jnp.dot(x_ref[...], w_ref[...], preferred_element_type=jnp.float32)
    h_ref[...] = h
    a_s = jnp.sum(h * asr_ref[...], axis=1, keepdims=True)
    a_d = jnp.sum(h * adr_ref[...], axis=1, keepdims=True)
    as_ref[...] = a_s
    ad_ref[...] = a_d
    cur = jnp.concatenate(
        [jnp.full((1, D), jnp.max(a_s)), jnp.full((1, D), jnp.max(a_d))], axis=0)

    @pl.when(i == 0)
    def _():
        mx_ref[...] = cur

    @pl.when(i != 0)
    def _():
        mx_ref[...] = jnp.maximum(mx_ref[...], cur)


def _tc1(x_pad, gat_W, att_src_row, att_dst_row):
    return pl.pallas_call(
        _tc1_body,
        grid=(GRID,),
        in_specs=[
            pl.BlockSpec((BLK, D), lambda i: (i, 0)),
            pl.BlockSpec((D, D), lambda i: (0, 0)),
            pl.BlockSpec((1, D), lambda i: (0, 0)),
            pl.BlockSpec((1, D), lambda i: (0, 0)),
        ],
        out_specs=[
            pl.BlockSpec((BLK, D), lambda i: (i, 0)),
            pl.BlockSpec((BLK, 1), lambda i: (i, 0)),
            pl.BlockSpec((BLK, 1), lambda i: (i, 0)),
            pl.BlockSpec((2, D), lambda i: (0, 0)),
        ],
        out_shape=[
            jax.ShapeDtypeStruct((NPAD, D), jnp.float32),
            jax.ShapeDtypeStruct((NPAD, 1), jnp.float32),
            jax.ShapeDtypeStruct((NPAD, 1), jnp.float32),
            jax.ShapeDtypeStruct((2, D), jnp.float32),
        ],
    )(x_pad, gat_W, att_src_row, att_dst_row)


def _tc2_body(h_ref, as_ref, ad_ref, mx_ref, un_ref, dc_ref,
              bias_ref, wroot_ref, x1_ref, root_ref):
    i = pl.program_id(0)
    shift = mx_ref[0, 0] + mx_ref[1, 0]
    z = as_ref[...] + ad_ref[...]
    ea = jnp.exp(jnp.maximum(z, 0.2 * z) - shift)
    den = dc_ref[...] + ea
    un = un_ref[...] + ea * h_ref[...]
    x1 = jnp.maximum(un / den + bias_ref[...], 0.0)
    rowid = lax.broadcasted_iota(jnp.int32, (BLK, 1), 0) + i * BLK
    x1 = jnp.where(rowid < N, x1, 0.0)
    x1_ref[...] = x1
    root_ref[...] = jnp.dot(x1, wroot_ref[...], preferred_element_type=jnp.float32)


def _tc2(h, as_col, ad_col, mx, un, dcol, gat_bias_row, gc_Wroot):
    return pl.pallas_call(
        _tc2_body,
        grid=(GRID,),
        in_specs=[
            pl.BlockSpec((BLK, D), lambda i: (i, 0)),
            pl.BlockSpec((BLK, 1), lambda i: (i, 0)),
            pl.BlockSpec((BLK, 1), lambda i: (i, 0)),
            pl.BlockSpec((2, D), lambda i: (0, 0)),
            pl.BlockSpec((BLK, D), lambda i: (i, 0)),
            pl.BlockSpec((BLK, 1), lambda i: (i, 0)),
            pl.BlockSpec((1, D), lambda i: (0, 0)),
            pl.BlockSpec((D, D), lambda i: (0, 0)),
        ],
        out_specs=[
            pl.BlockSpec((BLK, D), lambda i: (i, 0)),
            pl.BlockSpec((BLK, D), lambda i: (i, 0)),
        ],
        out_shape=[
            jax.ShapeDtypeStruct((NPAD, D), jnp.float32),
            jax.ShapeDtypeStruct((NPAD, D), jnp.float32),
        ],
    )(h, as_col, ad_col, mx, un, dcol, gat_bias_row, gc_Wroot)


def _tc3_body(ag_ref, root_ref, wrel_ref, gcb_ref, w1_ref, b1_ref,
              w2_ref, b2_ref, y_ref):
    x2 = jnp.dot(ag_ref[...], wrel_ref[...], preferred_element_type=jnp.float32)
    x2 = jnp.maximum(x2 + root_ref[...] + gcb_ref[...], 0.0)
    x3 = jnp.maximum(
        jnp.dot(x2, w1_ref[...], preferred_element_type=jnp.float32) + b1_ref[...],
        0.0)
    y_ref[...] = jnp.dot(x3, w2_ref[...], preferred_element_type=jnp.float32) + b2_ref[...]


def _tc3(ag, root, gc_Wrel, gcb_row, lin1_W, b1_row, lin2_W, b2_row):
    return pl.pallas_call(
        _tc3_body,
        grid=(GRID,),
        in_specs=[
            pl.BlockSpec((BLK, D), lambda i: (i, 0)),
            pl.BlockSpec((BLK, D), lambda i: (i, 0)),
            pl.BlockSpec((D, D), lambda i: (0, 0)),
            pl.BlockSpec((1, D), lambda i: (0, 0)),
            pl.BlockSpec((D, D), lambda i: (0, 0)),
            pl.BlockSpec((1, D), lambda i: (0, 0)),
            pl.BlockSpec((D, D), lambda i: (0, 0)),
            pl.BlockSpec((1, D), lambda i: (0, 0)),
        ],
        out_specs=pl.BlockSpec((BLK, D), lambda i: (i, 0)),
        out_shape=jax.ShapeDtypeStruct((NPAD, D), jnp.float32),
    )(ag, root, gc_Wrel, gcb_row, lin1_W, b1_row, lin2_W, b2_row)


def _sc_mesh():
    return plsc.VectorSubcoreMesh(core_axis_name="c", subcore_axis_name="s",
                                  num_cores=NC, num_subcores=NS)


def _zero_rows(rows):
    def _zr(j, _):
        for cc in range(8):
            rows[j, pl.ds(cc * LANES, LANES)] = jnp.zeros((LANES,), jnp.float32)
        return 0
    lax.fori_loop(0, CH, _zr, 0)


def _ecompute(asb, adb, bv, grow, e_buf):
    for j in range(CH // LANES):
        asv = asb[pl.ds(j * LANES, LANES)]
        adv = adb[pl.ds(j * LANES, LANES)]
        z = asv + adv
        z = jnp.maximum(z, 0.2 * z) - bv
        e = jnp.exp(z)
        pos = lax.iota(jnp.int32, LANES) + (grow * CH + j * LANES)
        e = jnp.where(pos < E, e, 0.0)
        e_buf[pl.ds(j * LANES, LANES)] = e


def _scale_rows(rows, e_buf):
    def scale(g, _):
        ev = e_buf[pl.ds(g * LANES, LANES)]
        for t in range(LANES):
            es = ev[t]
            j = g * LANES + t
            for cc in range(8):
                rows[j, pl.ds(cc * LANES, LANES)] = rows[j, pl.ds(cc * LANES, LANES)] * es
        return 0
    lax.fori_loop(0, CH // LANES, scale, 0)


def _sc_gat_body(src_hbm, dst_hbm, asrc_hbm, adst_hbm, h_hbm, bv_hbm,
                 un_out, den_out,
                 bv_v, idx_s2, idx_d2, as0, as1, ad0, ad1, eb0, eb1,
                 rows0, rows1, zbuf, un_sp, den_sp,
                 g0, g1, s0, s1, t0, t1, u0, u1):
    cid = lax.axis_index("c")
    sid = lax.axis_index("s")

    @pl.when(cid == 0)
    def _():
        rowstart = sid * NCROW0

        _zero_rows(rows0)

        def _zz(j, _):
            zbuf[pl.ds(j * LANES, LANES)] = jnp.zeros((LANES,), jnp.float32)
            return 0
        lax.fori_loop(0, RPT // LANES, _zz, 0)

        pltpu.sync_copy(zbuf, den_sp.at[pl.ds(RPT * sid, RPT)])
        for k in range(RPT // CH):
            pltpu.sync_copy(rows0, un_sp.at[pl.ds(RPT * sid + CH * k, CH)])
        plsc.subcore_barrier()

        pltpu.sync_copy(bv_hbm, bv_v)
        bv = bv_v[...]

        for seg in range(NSEG):
            segrow = rowstart + seg * SEGR
            pltpu.sync_copy(src_hbm.at[pl.ds(segrow, SEGR)], idx_s2)
            pltpu.sync_copy(dst_hbm.at[pl.ds(segrow, SEGR)], idx_d2)

            pltpu.async_copy(asrc_hbm.at[idx_s2.at[0]], as0, t0)
            pltpu.async_copy(adst_hbm.at[idx_d2.at[0]], ad0, u0)
            pltpu.async_copy(asrc_hbm.at[idx_s2.at[1]], as1, t1)
            pltpu.async_copy(adst_hbm.at[idx_d2.at[1]], ad1, u1)
            pltpu.async_copy(h_hbm.at[idx_s2.at[0]], rows0, g0)
            pltpu.async_copy(h_hbm.at[idx_s2.at[1]], rows1, g1)

            def pair(k, _):
                l0 = k * 2
                l1 = l0 + 1
                pltpu.make_async_copy(asrc_hbm.at[idx_s2.at[l0]], as0, t0).wait()
                pltpu.make_async_copy(adst_hbm.at[idx_d2.at[l0]], ad0, u0).wait()
                _ecompute(as0, ad0, bv, segrow + l0, eb0)
                pltpu.sync_copy(eb0, den_sp.at[idx_d2.at[l0]], add=True)

                @pl.when(k > 0)
                def _():
                    pltpu.make_async_copy(rows1, un_sp.at[idx_d2.at[l1]], s1).wait()
                    pltpu.async_copy(h_hbm.at[idx_s2.at[l1]], rows1, g1)

                pltpu.make_async_copy(h_hbm.at[idx_s2.at[l0]], rows0, g0).wait()
                _scale_rows(rows0, eb0)

                @pl.when(k < SEGP - 1)
                def _():
                    pltpu.async_copy(asrc_hbm.at[idx_s2.at[l0 + 2]], as0, t0)
                    pltpu.async_copy(adst_hbm.at[idx_d2.at[l0 + 2]], ad0, u0)

                pltpu.async_copy(rows0, un_sp.at[idx_d2.at[l0]], s0, add=True)

                pltpu.make_async_copy(asrc_hbm.at[idx_s2.at[l1]], as1, t1).wait()
                pltpu.make_async_copy(adst_hbm.at[idx_d2.at[l1]], ad1, u1).wait()
                _ecompute(as1, ad1, bv, segrow + l1, eb1)
                pltpu.sync_copy(eb1, den_sp.at[idx_d2.at[l1]], add=True)
                pltpu.make_async_copy(h_hbm.at[idx_s2.at[l1]], rows1, g1).wait()
                _scale_rows(rows1, eb1)
                pltpu.make_async_copy(rows0, un_sp.at[idx_d2.at[l0]], s0).wait()

                @pl.when(k < SEGP - 1)
                def _():
                    pltpu.async_copy(h_hbm.at[idx_s2.at[l0 + 2]], rows0, g0)
                    pltpu.async_copy(asrc_hbm.at[idx_s2.at[l1 + 2]], as1, t1)
                    pltpu.async_copy(adst_hbm.at[idx_d2.at[l1 + 2]], ad1, u1)

                pltpu.async_copy(rows1, un_sp.at[idx_d2.at[l1]], s1, add=True)
                return 0

            lax.fori_loop(0, SEGP, pair, 0)
            pltpu.make_async_copy(rows1, un_sp.at[idx_d2.at[SEGR - 1]], s1).wait()

        plsc.subcore_barrier()

        pltpu.sync_copy(un_sp.at[pl.ds(RPT * sid, RPT)],
                        un_out.at[pl.ds(RPT * sid, RPT)])
        pltpu.sync_copy(den_sp.at[pl.ds(RPT * sid, RPT)], den_out.at[sid])


def _sc_gat(srcp2, dstp2, asrc, adst, h, bvec):
    return pl.kernel(
        _sc_gat_body,
        out_type=(
            jax.ShapeDtypeStruct((NPAD, D), jnp.float32),
            jax.ShapeDtypeStruct((NS, RPT), jnp.float32),
        ),
        mesh=_sc_mesh(),
        compiler_params=pltpu.CompilerParams(needs_layout_passes=False),
        scratch_types=[
            pltpu.VMEM((LANES,), jnp.float32),
            pltpu.VMEM((SEGR, CH), jnp.int32),
            pltpu.VMEM((SEGR, CH), jnp.int32),
            pltpu.VMEM((CH,), jnp.float32),
            pltpu.VMEM((CH,), jnp.float32),
            pltpu.VMEM((CH,), jnp.float32),
            pltpu.VMEM((CH,), jnp.float32),
            pltpu.VMEM((CH,), jnp.float32),
            pltpu.VMEM((CH,), jnp.float32),
            pltpu.VMEM((CH, D), jnp.float32),
            pltpu.VMEM((CH, D), jnp.float32),
            pltpu.VMEM((RPT,), jnp.float32),
            pltpu.VMEM_SHARED((NPAD, D), jnp.float32),
            pltpu.VMEM_SHARED((NPAD,), jnp.float32),
            pltpu.SemaphoreType.DMA,
            pltpu.SemaphoreType.DMA,
            pltpu.SemaphoreType.DMA,
            pltpu.SemaphoreType.DMA,
            pltpu.SemaphoreType.DMA,
            pltpu.SemaphoreType.DMA,
            pltpu.SemaphoreType.DMA,
            pltpu.SemaphoreType.DMA,
        ],
    )(srcp2, dstp2, asrc, adst, h, bvec)


def _sc_agg_body(src_hbm, dst_hbm, x1_hbm, ag_out,
                 idx_s2, idx_d2, rows0, rows1, agg_sp, g0, g1, s0, s1):
    cid = lax.axis_index("c")
    sid = lax.axis_index("s")

    @pl.when(cid == 0)
    def _():
        rowstart = sid * NCROW0

        _zero_rows(rows0)
        for k in range(RPT // CH):
            pltpu.sync_copy(rows0, agg_sp.at[pl.ds(RPT * sid + CH * k, CH)])
        plsc.subcore_barrier()

        for seg in range(NSEG):
            segrow = rowstart + seg * SEGR
            pltpu.sync_copy(src_hbm.at[pl.ds(segrow, SEGR)], idx_s2)
            pltpu.sync_copy(dst_hbm.at[pl.ds(segrow, SEGR)], idx_d2)

            pltpu.async_copy(x1_hbm.at[idx_s2.at[0]], rows0, g0)
            pltpu.async_copy(x1_hbm.at[idx_s2.at[1]], rows1, g1)

            def pair(k, _):
                l0 = k * 2
                l1 = l0 + 1

                @pl.when(k > 0)
                def _():
                    pltpu.make_async_copy(rows1, agg_sp.at[idx_d2.at[l1]], s1).wait()
                    pltpu.async_copy(x1_hbm.at[idx_s2.at[l1]], rows1, g1)

                pltpu.make_async_copy(x1_hbm.at[idx_s2.at[l0]], rows0, g0).wait()
                pltpu.async_copy(rows0, agg_sp.at[idx_d2.at[l0]], s0, add=True)

                pltpu.make_async_copy(x1_hbm.at[idx_s2.at[l1]], rows1, g1).wait()
                pltpu.make_async_copy(rows0, agg_sp.at[idx_d2.at[l0]], s0).wait()

                @pl.when(k < SEGP - 1)
                def _():
                    pltpu.async_copy(x1_hbm.at[idx_s2.at[l0 + 2]], rows0, g0)

                pltpu.async_copy(rows1, agg_sp.at[idx_d2.at[l1]], s1, add=True)
                return 0

            lax.fori_loop(0, SEGP, pair, 0)
            pltpu.make_async_copy(rows1, agg_sp.at[idx_d2.at[SEGR - 1]], s1).wait()

        plsc.subcore_barrier()
        pltpu.sync_copy(agg_sp.at[pl.ds(RPT * sid, RPT)],
                        ag_out.at[pl.ds(RPT * sid, RPT)])


def _sc_agg(srcp2, dstp2, x1):
    return pl.kernel(
        _sc_agg_body,
        out_type=jax.ShapeDtypeStruct((NPAD, D), jnp.float32),
        mesh=_sc_mesh(),
        compiler_params=pltpu.CompilerParams(needs_layout_passes=False),
        scratch_types=[
            pltpu.VMEM((SEGR, CH), jnp.int32),
            pltpu.VMEM((SEGR, CH), jnp.int32),
            pltpu.VMEM((CH, D), jnp.float32),
            pltpu.VMEM((CH, D), jnp.float32),
            pltpu.VMEM_SHARED((NPAD, D), jnp.float32),
            pltpu.SemaphoreType.DMA,
            pltpu.SemaphoreType.DMA,
            pltpu.SemaphoreType.DMA,
            pltpu.SemaphoreType.DMA,
        ],
    )(srcp2, dstp2, x1)


@jax.jit
def kernel(x, edge_index, edge_attr, batch, gat_W, gat_att_src, gat_att_dst,
           gat_bias, gc_Wrel, gc_Wroot, gc_bias, lin1_W, lin1_b, lin2_W,
           lin2_b):
    x_pad = jnp.zeros((NPAD, D), jnp.float32).at[:N].set(x)
    srcp = jnp.concatenate([
        edge_index[0].astype(jnp.int32),
        jnp.full((EPAD - E,), N, jnp.int32),
    ]).reshape(EPAD // CH, CH)
    dstp = jnp.concatenate([
        edge_index[1].astype(jnp.int32),
        jnp.zeros((EPAD - E,), jnp.int32),
    ]).reshape(EPAD // CH, CH)

    h, as_col, ad_col, mx = _tc1(x_pad, gat_W,
                                 gat_att_src.reshape(1, D),
                                 gat_att_dst.reshape(1, D))
    bvec = jnp.full((LANES,), mx[0, 0] + mx[1, 0], jnp.float32)

    un, den = _sc_gat(srcp, dstp, as_col.reshape(NPAD), ad_col.reshape(NPAD),
                      h, bvec)
    dcol = den.reshape(NPAD, 1)

    x1, root = _tc2(h, as_col, ad_col, mx, un, dcol,
                    gat_bias.reshape(1, D), gc_Wroot)

    ag = _sc_agg(srcp, dstp, x1)

    y = _tc3(ag, root, gc_Wrel, gc_bias.reshape(1, D),
             lin1_W, lin1_b.reshape(1, D), lin2_W, lin2_b.reshape(1, D))
    return y[:N]


# speedup vs baseline: 1.4032x; 1.3375x over previous
import jax
import jax.numpy as jnp
from jax import lax
from jax.experimental import pallas as pl
from jax.experimental.pallas import tpu as pltpu
from jax.experimental.pallas import tpu_sc as plsc

N = 10000
NPAD = 10240
D = 128
E = 320000
NC, NS, LANES = 2, 16, 16
NW = NC * NS
CH = 128
EPW = 10112
NCHUNK = EPW // CH
EPAD = NW * EPW
RPT = NPAD // NS
BLK = 1024
GRID = NPAD // BLK


def _tc1_body(x_ref, w_ref, asr_ref, adr_ref, h_ref, as_ref, ad_ref, mx_ref):
    i = pl.program_id(0)
    h = jnp.dot(x_ref[...], w_ref[...], preferred_element_type=jnp.float32)
    h_ref[...] = h
    a_s = jnp.sum(h * asr_ref[...], axis=1, keepdims=True)
    a_d = jnp.sum(h * adr_ref[...], axis=1, keepdims=True)
    as_ref[...] = a_s
    ad_ref[...] = a_d
    cur = jnp.concatenate(
        [jnp.full((1, D), jnp.max(a_s)), jnp.full((1, D), jnp.max(a_d))], axis=0)

    @pl.when(i == 0)
    def _():
        mx_ref[...] = cur

    @pl.when(i != 0)
    def _():
        mx_ref[...] = jnp.maximum(mx_ref[...], cur)


def _tc1(x_pad, gat_W, att_src_row, att_dst_row):
    return pl.pallas_call(
        _tc1_body,
        grid=(GRID,),
        in_specs=[
            pl.BlockSpec((BLK, D), lambda i: (i, 0)),
            pl.BlockSpec((D, D), lambda i: (0, 0)),
            pl.BlockSpec((1, D), lambda i: (0, 0)),
            pl.BlockSpec((1, D), lambda i: (0, 0)),
        ],
        out_specs=[
            pl.BlockSpec((BLK, D), lambda i: (i, 0)),
            pl.BlockSpec((BLK, 1), lambda i: (i, 0)),
            pl.BlockSpec((BLK, 1), lambda i: (i, 0)),
            pl.BlockSpec((2, D), lambda i: (0, 0)),
        ],
        out_shape=[
            jax.ShapeDtypeStruct((NPAD, D), jnp.float32),
            jax.ShapeDtypeStruct((NPAD, 1), jnp.float32),
            jax.ShapeDtypeStruct((NPAD, 1), jnp.float32),
            jax.ShapeDtypeStruct((2, D), jnp.float32),
        ],
    )(x_pad, gat_W, att_src_row, att_dst_row)


def _tc2_body(h_ref, as_ref, ad_ref, mx_ref, un0_ref, un1_ref, dc_ref,
              bias_ref, wroot_ref, x1_ref, root_ref):
    i = pl.program_id(0)
    shift = mx_ref[0, 0] + mx_ref[1, 0]
    z = as_ref[...] + ad_ref[...]
    ea = jnp.exp(jnp.maximum(z, 0.2 * z) - shift)
    den = dc_ref[:, 0:1] + dc_ref[:, 1:2] + ea
    un = un0_ref[...] + un1_ref[...] + ea * h_ref[...]
    x1 = jnp.maximum(un / den + bias_ref[...], 0.0)
    rowid = lax.broadcasted_iota(jnp.int32, (BLK, 1), 0) + i * BLK
    x1 = jnp.where(rowid < N, x1, 0.0)
    x1_ref[...] = x1
    root_ref[...] = jnp.dot(x1, wroot_ref[...], preferred_element_type=jnp.float32)


def _tc2(h, as_col, ad_col, mx, un0, un1, dcol, gat_bias_row, gc_Wroot):
    return pl.pallas_call(
        _tc2_body,
        grid=(GRID,),
        in_specs=[
            pl.BlockSpec((BLK, D), lambda i: (i, 0)),
            pl.BlockSpec((BLK, 1), lambda i: (i, 0)),
            pl.BlockSpec((BLK, 1), lambda i: (i, 0)),
            pl.BlockSpec((2, D), lambda i: (0, 0)),
            pl.BlockSpec((BLK, D), lambda i: (i, 0)),
            pl.BlockSpec((BLK, D), lambda i: (i, 0)),
            pl.BlockSpec((BLK, 2), lambda i: (i, 0)),
            pl.BlockSpec((1, D), lambda i: (0, 0)),
            pl.BlockSpec((D, D), lambda i: (0, 0)),
        ],
        out_specs=[
            pl.BlockSpec((BLK, D), lambda i: (i, 0)),
            pl.BlockSpec((BLK, D), lambda i: (i, 0)),
        ],
        out_shape=[
            jax.ShapeDtypeStruct((NPAD, D), jnp.float32),
            jax.ShapeDtypeStruct((NPAD, D), jnp.float32),
        ],
    )(h, as_col, ad_col, mx, un0, un1, dcol, gat_bias_row, gc_Wroot)


def _tc3_body(ag0_ref, ag1_ref, root_ref, wrel_ref, gcb_ref, w1_ref, b1_ref,
              w2_ref, b2_ref, y_ref):
    agg = ag0_ref[...] + ag1_ref[...]
    x2 = jnp.dot(agg, wrel_ref[...], preferred_element_type=jnp.float32)
    x2 = jnp.maximum(x2 + root_ref[...] + gcb_ref[...], 0.0)
    x3 = jnp.maximum(
        jnp.dot(x2, w1_ref[...], preferred_element_type=jnp.float32) + b1_ref[...],
        0.0)
    y_ref[...] = jnp.dot(x3, w2_ref[...], preferred_element_type=jnp.float32) + b2_ref[...]


def _tc3(ag0, ag1, root, gc_Wrel, gcb_row, lin1_W, b1_row, lin2_W, b2_row):
    return pl.pallas_call(
        _tc3_body,
        grid=(GRID,),
        in_specs=[
            pl.BlockSpec((BLK, D), lambda i: (i, 0)),
            pl.BlockSpec((BLK, D), lambda i: (i, 0)),
            pl.BlockSpec((BLK, D), lambda i: (i, 0)),
            pl.BlockSpec((D, D), lambda i: (0, 0)),
            pl.BlockSpec((1, D), lambda i: (0, 0)),
            pl.BlockSpec((D, D), lambda i: (0, 0)),
            pl.BlockSpec((1, D), lambda i: (0, 0)),
            pl.BlockSpec((D, D), lambda i: (0, 0)),
            pl.BlockSpec((1, D), lambda i: (0, 0)),
        ],
        out_specs=pl.BlockSpec((BLK, D), lambda i: (i, 0)),
        out_shape=jax.ShapeDtypeStruct((NPAD, D), jnp.float32),
    )(ag0, ag1, root, gc_Wrel, gcb_row, lin1_W, b1_row, lin2_W, b2_row)


def _sc_mesh():
    return plsc.VectorSubcoreMesh(core_axis_name="c", subcore_axis_name="s",
                                  num_cores=NC, num_subcores=NS)


def _sc_gat_body(src_hbm, dst_hbm, asrc_hbm, adst_hbm, h_hbm, bv_hbm,
                 un_out, den_out,
                 asrc_v, adst_v, bv_v, idx_s, idx_d, e_buf, rows, zbuf,
                 un_sp, den_sp, gsem):
    cid = lax.axis_index("c")
    sid = lax.axis_index("s")
    wid = sid * NC + cid

    def _zr(j, _):
        for cc in range(8):
            rows[j, pl.ds(cc * LANES, LANES)] = jnp.zeros((LANES,), jnp.float32)
        return 0
    lax.fori_loop(0, CH, _zr, 0)

    def _zz(j, _):
        zbuf[pl.ds(j * LANES, LANES)] = jnp.zeros((LANES,), jnp.float32)
        return 0
    lax.fori_loop(0, RPT // LANES, _zz, 0)

    pltpu.sync_copy(zbuf, den_sp.at[pl.ds(RPT * sid, RPT)])
    for k in range(RPT // CH):
        pltpu.sync_copy(rows, un_sp.at[pl.ds(RPT * sid + CH * k, CH)])
    plsc.subcore_barrier()

    pltpu.sync_copy(asrc_hbm, asrc_v)
    pltpu.sync_copy(adst_hbm, adst_v)
    pltpu.sync_copy(bv_hbm, bv_v)
    bv = bv_v[...]

    base0 = wid * EPW

    def chunk(c, _):
        base = pl.multiple_of(base0 + c * CH, CH)
        pltpu.sync_copy(src_hbm.at[pl.ds(base, CH)], idx_s)
        pltpu.sync_copy(dst_hbm.at[pl.ds(base, CH)], idx_d)
        for j in range(CH // LANES):
            isv = idx_s[pl.ds(j * LANES, LANES)]
            idv = idx_d[pl.ds(j * LANES, LANES)]
            asv = plsc.load_gather(asrc_v, [isv])
            adv = plsc.load_gather(adst_v, [idv])
            z = asv + adv
            z = jnp.maximum(z, 0.2 * z) - bv
            e = jnp.exp(z)
            pos = lax.iota(jnp.int32, LANES) + (base + j * LANES)
            e = jnp.where(pos < E, e, 0.0)
            e_buf[pl.ds(j * LANES, LANES)] = e
        pltpu.sync_copy(e_buf, den_sp.at[idx_d], add=True)
        pltpu.async_copy(h_hbm.at[idx_s], rows, gsem).wait()

        def scale(g, _):
            ev = e_buf[pl.ds(g * LANES, LANES)]
            for t in range(LANES):
                es = ev[t]
                j = g * LANES + t
                for cc in range(8):
                    rows[j, pl.ds(cc * LANES, LANES)] = rows[j, pl.ds(cc * LANES, LANES)] * es
            return 0
        lax.fori_loop(0, CH // LANES, scale, 0)
        pltpu.sync_copy(rows, un_sp.at[idx_d], add=True)
        return 0

    lax.fori_loop(0, NCHUNK, chunk, 0)
    plsc.subcore_barrier()

    pltpu.sync_copy(un_sp.at[pl.ds(RPT * sid, RPT)],
                    un_out.at[cid, pl.ds(RPT * sid, RPT)])
    pltpu.sync_copy(den_sp.at[pl.ds(RPT * sid, RPT)], den_out.at[cid, sid])


def _sc_gat(srcp, dstp, asrc, adst, h, bvec):
    return pl.kernel(
        _sc_gat_body,
        out_type=(
            jax.ShapeDtypeStruct((NC, NPAD, D), jnp.float32),
            jax.ShapeDtypeStruct((NC, NS, RPT), jnp.float32),
        ),
        mesh=_sc_mesh(),
        compiler_params=pltpu.CompilerParams(needs_layout_passes=False),
        scratch_types=[
            pltpu.VMEM((NPAD,), jnp.float32),
            pltpu.VMEM((NPAD,), jnp.float32),
            pltpu.VMEM((LANES,), jnp.float32),
            pltpu.VMEM((CH,), jnp.int32),
            pltpu.VMEM((CH,), jnp.int32),
            pltpu.VMEM((CH,), jnp.float32),
            pltpu.VMEM((CH, D), jnp.float32),
            pltpu.VMEM((RPT,), jnp.float32),
            pltpu.VMEM_SHARED((NPAD, D), jnp.float32),
            pltpu.VMEM_SHARED((NPAD,), jnp.float32),
            pltpu.SemaphoreType.DMA,
        ],
    )(srcp, dstp, asrc, adst, h, bvec)


def _sc_agg_body(src_hbm, dst_hbm, x1_hbm, ag_out,
                 idx_s, idx_d, rows, agg_sp, gsem):
    cid = lax.axis_index("c")
    sid = lax.axis_index("s")
    wid = sid * NC + cid

    def _zr(j, _):
        for cc in range(8):
            rows[j, pl.ds(cc * LANES, LANES)] = jnp.zeros((LANES,), jnp.float32)
        return 0
    lax.fori_loop(0, CH, _zr, 0)
    for k in range(RPT // CH):
        pltpu.sync_copy(rows, agg_sp.at[pl.ds(RPT * sid + CH * k, CH)])
    plsc.subcore_barrier()

    base0 = wid * EPW

    def chunk(c, _):
        base = pl.multiple_of(base0 + c * CH, CH)
        pltpu.sync_copy(src_hbm.at[pl.ds(base, CH)], idx_s)
        pltpu.sync_copy(dst_hbm.at[pl.ds(base, CH)], idx_d)
        pltpu.async_copy(x1_hbm.at[idx_s], rows, gsem).wait()
        pltpu.sync_copy(rows, agg_sp.at[idx_d], add=True)
        return 0

    lax.fori_loop(0, NCHUNK, chunk, 0)
    plsc.subcore_barrier()
    pltpu.sync_copy(agg_sp.at[pl.ds(RPT * sid, RPT)],
                    ag_out.at[cid, pl.ds(RPT * sid, RPT)])


def _sc_agg(srcp, dstp, x1):
    return pl.kernel(
        _sc_agg_body,
        out_type=jax.ShapeDtypeStruct((NC, NPAD, D), jnp.float32),
        mesh=_sc_mesh(),
        compiler_params=pltpu.CompilerParams(needs_layout_passes=False),
        scratch_types=[
            pltpu.VMEM((CH,), jnp.int32),
            pltpu.VMEM((CH,), jnp.int32),
            pltpu.VMEM((CH, D), jnp.float32),
            pltpu.VMEM_SHARED((NPAD, D), jnp.float32),
            pltpu.SemaphoreType.DMA,
        ],
    )(srcp, dstp, x1)


@jax.jit
def kernel(x, edge_index, edge_attr, batch, gat_W, gat_att_src, gat_att_dst,
           gat_bias, gc_Wrel, gc_Wroot, gc_bias, lin1_W, lin1_b, lin2_W,
           lin2_b):
    x_pad = jnp.zeros((NPAD, D), jnp.float32).at[:N].set(x)
    srcp = jnp.concatenate([
        edge_index[0].astype(jnp.int32),
        jnp.full((EPAD - E,), N, jnp.int32),
    ])
    dstp = jnp.concatenate([
        edge_index[1].astype(jnp.int32),
        jnp.zeros((EPAD - E,), jnp.int32),
    ])

    h, as_col, ad_col, mx = _tc1(x_pad, gat_W,
                                 gat_att_src.reshape(1, D),
                                 gat_att_dst.reshape(1, D))
    bvec = jnp.full((LANES,), mx[0, 0] + mx[1, 0], jnp.float32)

    un_p, den_p = _sc_gat(srcp, dstp, as_col.reshape(NPAD), ad_col.reshape(NPAD),
                          h, bvec)
    dcol = den_p.reshape(NC, NPAD).T

    x1, root = _tc2(h, as_col, ad_col, mx, un_p[0], un_p[1], dcol,
                    gat_bias.reshape(1, D), gc_Wroot)

    ag_p = _sc_agg(srcp, dstp, x1)

    y = _tc3(ag_p[0], ag_p[1], root, gc_Wrel, gc_bias.reshape(1, D),
             lin1_W, lin1_b.reshape(1, D), lin2_W, lin2_b.reshape(1, D))
    return y[:N]


# speedup vs baseline: 1.4817x; 1.0560x over previous
import jax
import jax.numpy as jnp
from jax import lax
from jax.experimental import pallas as pl
from jax.experimental.pallas import tpu as pltpu
from jax.experimental.pallas import tpu_sc as plsc

N = 10000
NPAD = 10240
D = 128
E = 320000
NC, NS, LANES = 2, 16, 16
NW = NC * NS
CH = 128
EPW = 10112
NCHUNK = EPW // CH
EPAD = NW * EPW
RPT = NPAD // NS
BLK = 1024
GRID = NPAD // BLK


def _tc1_body(x_ref, w_ref, asr_ref, adr_ref, h_ref, as_ref, ad_ref, mx_ref):
    i = pl.program_id(0)
    h = jnp.dot(x_ref[...], w_ref[...], preferred_element_type=jnp.float32)
    h_ref[...] = h
    a_s = jnp.sum(h * asr_ref[...], axis=1, keepdims=True)
    a_d = jnp.sum(h * adr_ref[...], axis=1, keepdims=True)
    as_ref[...] = a_s
    ad_ref[...] = a_d
    cur = jnp.concatenate(
        [jnp.full((1, D), jnp.max(a_s)), jnp.full((1, D), jnp.max(a_d))], axis=0)

    @pl.when(i == 0)
    def _():
        mx_ref[...] = cur

    @pl.when(i != 0)
    def _():
        mx_ref[...] = jnp.maximum(mx_ref[...], cur)


def _tc1(x_pad, gat_W, att_src_row, att_dst_row):
    return pl.pallas_call(
        _tc1_body,
        grid=(GRID,),
        in_specs=[
            pl.BlockSpec((BLK, D), lambda i: (i, 0)),
            pl.BlockSpec((D, D), lambda i: (0, 0)),
            pl.BlockSpec((1, D), lambda i: (0, 0)),
            pl.BlockSpec((1, D), lambda i: (0, 0)),
        ],
        out_specs=[
            pl.BlockSpec((BLK, D), lambda i: (i, 0)),
            pl.BlockSpec((BLK, 1), lambda i: (i, 0)),
            pl.BlockSpec((BLK, 1), lambda i: (i, 0)),
            pl.BlockSpec((2, D), lambda i: (0, 0)),
        ],
        out_shape=[
            jax.ShapeDtypeStruct((NPAD, D), jnp.float32),
            jax.ShapeDtypeStruct((NPAD, 1), jnp.float32),
            jax.ShapeDtypeStruct((NPAD, 1), jnp.float32),
            jax.ShapeDtypeStruct((2, D), jnp.float32),
        ],
    )(x_pad, gat_W, att_src_row, att_dst_row)


def _tc2_body(h_ref, as_ref, ad_ref, mx_ref, un0_ref, un1_ref, dc_ref,
              bias_ref, wroot_ref, x1_ref, root_ref):
    i = pl.program_id(0)
    shift = mx_ref[0, 0] + mx_ref[1, 0]
    z = as_ref[...] + ad_ref[...]
    ea = jnp.exp(jnp.maximum(z, 0.2 * z) - shift)
    den = dc_ref[:, 0:1] + dc_ref[:, 1:2] + ea
    un = un0_ref[...] + un1_ref[...] + ea * h_ref[...]
    x1 = jnp.maximum(un / den + bias_ref[...], 0.0)
    rowid = lax.broadcasted_iota(jnp.int32, (BLK, 1), 0) + i * BLK
    x1 = jnp.where(rowid < N, x1, 0.0)
    x1_ref[...] = x1
    root_ref[...] = jnp.dot(x1, wroot_ref[...], preferred_element_type=jnp.float32)


def _tc2(h, as_col, ad_col, mx, un0, un1, dcol, gat_bias_row, gc_Wroot):
    return pl.pallas_call(
        _tc2_body,
        grid=(GRID,),
        in_specs=[
            pl.BlockSpec((BLK, D), lambda i: (i, 0)),
            pl.BlockSpec((BLK, 1), lambda i: (i, 0)),
            pl.BlockSpec((BLK, 1), lambda i: (i, 0)),
            pl.BlockSpec((2, D), lambda i: (0, 0)),
            pl.BlockSpec((BLK, D), lambda i: (i, 0)),
            pl.BlockSpec((BLK, D), lambda i: (i, 0)),
            pl.BlockSpec((BLK, 2), lambda i: (i, 0)),
            pl.BlockSpec((1, D), lambda i: (0, 0)),
            pl.BlockSpec((D, D), lambda i: (0, 0)),
        ],
        out_specs=[
            pl.BlockSpec((BLK, D), lambda i: (i, 0)),
            pl.BlockSpec((BLK, D), lambda i: (i, 0)),
        ],
        out_shape=[
            jax.ShapeDtypeStruct((NPAD, D), jnp.float32),
            jax.ShapeDtypeStruct((NPAD, D), jnp.float32),
        ],
    )(h, as_col, ad_col, mx, un0, un1, dcol, gat_bias_row, gc_Wroot)


def _tc3_body(ag0_ref, ag1_ref, root_ref, wrel_ref, gcb_ref, w1_ref, b1_ref,
              w2_ref, b2_ref, y_ref):
    agg = ag0_ref[...] + ag1_ref[...]
    x2 = jnp.dot(agg, wrel_ref[...], preferred_element_type=jnp.float32)
    x2 = jnp.maximum(x2 + root_ref[...] + gcb_ref[...], 0.0)
    x3 = jnp.maximum(
        jnp.dot(x2, w1_ref[...], preferred_element_type=jnp.float32) + b1_ref[...],
        0.0)
    y_ref[...] = jnp.dot(x3, w2_ref[...], preferred_element_type=jnp.float32) + b2_ref[...]


def _tc3(ag0, ag1, root, gc_Wrel, gcb_row, lin1_W, b1_row, lin2_W, b2_row):
    return pl.pallas_call(
        _tc3_body,
        grid=(GRID,),
        in_specs=[
            pl.BlockSpec((BLK, D), lambda i: (i, 0)),
            pl.BlockSpec((BLK, D), lambda i: (i, 0)),
            pl.BlockSpec((BLK, D), lambda i: (i, 0)),
            pl.BlockSpec((D, D), lambda i: (0, 0)),
            pl.BlockSpec((1, D), lambda i: (0, 0)),
            pl.BlockSpec((D, D), lambda i: (0, 0)),
            pl.BlockSpec((1, D), lambda i: (0, 0)),
            pl.BlockSpec((D, D), lambda i: (0, 0)),
            pl.BlockSpec((1, D), lambda i: (0, 0)),
        ],
        out_specs=pl.BlockSpec((BLK, D), lambda i: (i, 0)),
        out_shape=jax.ShapeDtypeStruct((NPAD, D), jnp.float32),
    )(ag0, ag1, root, gc_Wrel, gcb_row, lin1_W, b1_row, lin2_W, b2_row)


def _sc_mesh():
    return plsc.VectorSubcoreMesh(core_axis_name="c", subcore_axis_name="s",
                                  num_cores=NC, num_subcores=NS)


def _sc_gat_body(src_hbm, dst_hbm, asrc_hbm, adst_hbm, h_hbm, bv_hbm,
                 un_out, den_out,
                 asrc_v, adst_v, bv_v, idx_s, idx_d, e_buf, rows, zbuf,
                 un_sp, den_sp, gsem):
    cid = lax.axis_index("c")
    sid = lax.axis_index("s")
    wid = sid * NC + cid

    def _zr(j, _):
        for cc in range(8):
            rows[j, pl.ds(cc * LANES, LANES)] = jnp.zeros((LANES,), jnp.float32)
        return 0
    lax.fori_loop(0, CH, _zr, 0)

    def _zz(j, _):
        zbuf[pl.ds(j * LANES, LANES)] = jnp.zeros((LANES,), jnp.float32)
        return 0
    lax.fori_loop(0, RPT // LANES, _zz, 0)

    pltpu.sync_copy(zbuf, den_sp.at[pl.ds(RPT * sid, RPT)])
    for k in range(RPT // CH):
        pltpu.sync_copy(rows, un_sp.at[pl.ds(RPT * sid + CH * k, CH)])
    plsc.subcore_barrier()

    pltpu.sync_copy(asrc_hbm, asrc_v)
    pltpu.sync_copy(adst_hbm, adst_v)
    pltpu.sync_copy(bv_hbm, bv_v)
    bv = bv_v[...]

    base0 = wid * EPW

    def chunk(c, _):
        base = pl.multiple_of(base0 + c * CH, CH)
        pltpu.sync_copy(src_hbm.at[pl.ds(base, CH)], idx_s)
        pltpu.sync_copy(dst_hbm.at[pl.ds(base, CH)], idx_d)
        gcopy = pltpu.async_copy(h_hbm.at[idx_s], rows, gsem)
        for j in range(CH // LANES):
            isv = idx_s[pl.ds(j * LANES, LANES)]
            idv = idx_d[pl.ds(j * LANES, LANES)]
            asv = plsc.load_gather(asrc_v, [isv])
            adv = plsc.load_gather(adst_v, [idv])
            z = asv + adv
            z = jnp.maximum(z, 0.2 * z) - bv
            e = jnp.exp(z)
            pos = lax.iota(jnp.int32, LANES) + (base + j * LANES)
            e = jnp.where(pos < E, e, 0.0)
            e_buf[pl.ds(j * LANES, LANES)] = e
        pltpu.sync_copy(e_buf, den_sp.at[idx_d], add=True)
        gcopy.wait()

        def scale(g, _):
            ev = e_buf[pl.ds(g * LANES, LANES)]
            for t in range(LANES):
                es = ev[t]
                j = g * LANES + t
                for cc in range(8):
                    rows[j, pl.ds(cc * LANES, LANES)] = rows[j, pl.ds(cc * LANES, LANES)] * es
            return 0
        lax.fori_loop(0, CH // LANES, scale, 0)
        pltpu.sync_copy(rows, un_sp.at[idx_d], add=True)
        return 0

    lax.fori_loop(0, NCHUNK, chunk, 0)
    plsc.subcore_barrier()

    pltpu.sync_copy(un_sp.at[pl.ds(RPT * sid, RPT)],
                    un_out.at[cid, pl.ds(RPT * sid, RPT)])
    pltpu.sync_copy(den_sp.at[pl.ds(RPT * sid, RPT)], den_out.at[cid, sid])


def _sc_gat(srcp, dstp, asrc, adst, h, bvec):
    return pl.kernel(
        _sc_gat_body,
        out_type=(
            jax.ShapeDtypeStruct((NC, NPAD, D), jnp.float32),
            jax.ShapeDtypeStruct((NC, NS, RPT), jnp.float32),
        ),
        mesh=_sc_mesh(),
        compiler_params=pltpu.CompilerParams(needs_layout_passes=False),
        scratch_types=[
            pltpu.VMEM((NPAD,), jnp.float32),
            pltpu.VMEM((NPAD,), jnp.float32),
            pltpu.VMEM((LANES,), jnp.float32),
            pltpu.VMEM((CH,), jnp.int32),
            pltpu.VMEM((CH,), jnp.int32),
            pltpu.VMEM((CH,), jnp.float32),
            pltpu.VMEM((CH, D), jnp.float32),
            pltpu.VMEM((RPT,), jnp.float32),
            pltpu.VMEM_SHARED((NPAD, D), jnp.float32),
            pltpu.VMEM_SHARED((NPAD,), jnp.float32),
            pltpu.SemaphoreType.DMA,
        ],
    )(srcp, dstp, asrc, adst, h, bvec)


def _sc_agg_body(src_hbm, dst_hbm, x1_hbm, ag_out,
                 idx_s, idx_d, rows, agg_sp, gsem):
    cid = lax.axis_index("c")
    sid = lax.axis_index("s")
    wid = sid * NC + cid

    def _zr(j, _):
        for cc in range(8):
            rows[j, pl.ds(cc * LANES, LANES)] = jnp.zeros((LANES,), jnp.float32)
        return 0
    lax.fori_loop(0, CH, _zr, 0)
    for k in range(RPT // CH):
        pltpu.sync_copy(rows, agg_sp.at[pl.ds(RPT * sid + CH * k, CH)])
    plsc.subcore_barrier()

    base0 = wid * EPW

    def chunk(c, _):
        base = pl.multiple_of(base0 + c * CH, CH)
        pltpu.sync_copy(src_hbm.at[pl.ds(base, CH)], idx_s)
        gcopy = pltpu.async_copy(x1_hbm.at[idx_s], rows, gsem)
        pltpu.sync_copy(dst_hbm.at[pl.ds(base, CH)], idx_d)
        gcopy.wait()
        pltpu.sync_copy(rows, agg_sp.at[idx_d], add=True)
        return 0

    lax.fori_loop(0, NCHUNK, chunk, 0)
    plsc.subcore_barrier()
    pltpu.sync_copy(agg_sp.at[pl.ds(RPT * sid, RPT)],
                    ag_out.at[cid, pl.ds(RPT * sid, RPT)])


def _sc_agg(srcp, dstp, x1):
    return pl.kernel(
        _sc_agg_body,
        out_type=jax.ShapeDtypeStruct((NC, NPAD, D), jnp.float32),
        mesh=_sc_mesh(),
        compiler_params=pltpu.CompilerParams(needs_layout_passes=False),
        scratch_types=[
            pltpu.VMEM((CH,), jnp.int32),
            pltpu.VMEM((CH,), jnp.int32),
            pltpu.VMEM((CH, D), jnp.float32),
            pltpu.VMEM_SHARED((NPAD, D), jnp.float32),
            pltpu.SemaphoreType.DMA,
        ],
    )(srcp, dstp, x1)


@jax.jit
def kernel(x, edge_index, edge_attr, batch, gat_W, gat_att_src, gat_att_dst,
           gat_bias, gc_Wrel, gc_Wroot, gc_bias, lin1_W, lin1_b, lin2_W,
           lin2_b):
    x_pad = jnp.zeros((NPAD, D), jnp.float32).at[:N].set(x)
    srcp = jnp.concatenate([
        edge_index[0].astype(jnp.int32),
        jnp.full((EPAD - E,), N, jnp.int32),
    ])
    dstp = jnp.concatenate([
        edge_index[1].astype(jnp.int32),
        jnp.zeros((EPAD - E,), jnp.int32),
    ])

    h, as_col, ad_col, mx = _tc1(x_pad, gat_W,
                                 gat_att_src.reshape(1, D),
                                 gat_att_dst.reshape(1, D))
    bvec = jnp.full((LANES,), mx[0, 0] + mx[1, 0], jnp.float32)

    un_p, den_p = _sc_gat(srcp, dstp, as_col.reshape(NPAD), ad_col.reshape(NPAD),
                          h, bvec)
    dcol = den_p.reshape(NC, NPAD).T

    x1, root = _tc2(h, as_col, ad_col, mx, un_p[0], un_p[1], dcol,
                    gat_bias.reshape(1, D), gc_Wroot)

    ag_p = _sc_agg(srcp, dstp, x1)

    y = _tc3(ag_p[0], ag_p[1], root, gc_Wrel, gc_bias.reshape(1, D),
             lin1_W, lin1_b.reshape(1, D), lin2_W, lin2_b.reshape(1, D))
    return y[:N]


# speedup vs baseline: 1.5831x; 1.0684x over previous
import jax
import jax.numpy as jnp
from jax import lax
from jax.experimental import pallas as pl
from jax.experimental.pallas import tpu as pltpu
from jax.experimental.pallas import tpu_sc as plsc

N = 10000
NPAD = 10240
D = 128
E = 320000
NC, NS, LANES = 2, 16, 16
NW = NC * NS
CH = 128
EPW = 10112
NCHUNK = EPW // CH
EPAD = NW * EPW
C0, C1 = 90, 68
RPT = NPAD // NS
BLK = 1024
GRID = NPAD // BLK


def _tc1_body(x_ref, w_ref, asr_ref, adr_ref, h_ref, as_ref, ad_ref, mx_ref):
    i = pl.program_id(0)
    h = jnp.dot(x_ref[...], w_ref[...], preferred_element_type=jnp.float32)
    h_ref[...] = h
    a_s = jnp.sum(h * asr_ref[...], axis=1, keepdims=True)
    a_d = jnp.sum(h * adr_ref[...], axis=1, keepdims=True)
    as_ref[...] = a_s
    ad_ref[...] = a_d
    cur = jnp.concatenate(
        [jnp.full((1, D), jnp.max(a_s)), jnp.full((1, D), jnp.max(a_d))], axis=0)

    @pl.when(i == 0)
    def _():
        mx_ref[...] = cur

    @pl.when(i != 0)
    def _():
        mx_ref[...] = jnp.maximum(mx_ref[...], cur)


def _tc1(x_pad, gat_W, att_src_row, att_dst_row):
    return pl.pallas_call(
        _tc1_body,
        grid=(GRID,),
        in_specs=[
            pl.BlockSpec((BLK, D), lambda i: (i, 0)),
            pl.BlockSpec((D, D), lambda i: (0, 0)),
            pl.BlockSpec((1, D), lambda i: (0, 0)),
            pl.BlockSpec((1, D), lambda i: (0, 0)),
        ],
        out_specs=[
            pl.BlockSpec((BLK, D), lambda i: (i, 0)),
            pl.BlockSpec((BLK, 1), lambda i: (i, 0)),
            pl.BlockSpec((BLK, 1), lambda i: (i, 0)),
            pl.BlockSpec((2, D), lambda i: (0, 0)),
        ],
        out_shape=[
            jax.ShapeDtypeStruct((NPAD, D), jnp.float32),
            jax.ShapeDtypeStruct((NPAD, 1), jnp.float32),
            jax.ShapeDtypeStruct((NPAD, 1), jnp.float32),
            jax.ShapeDtypeStruct((2, D), jnp.float32),
        ],
    )(x_pad, gat_W, att_src_row, att_dst_row)


def _tc2_body(h_ref, as_ref, ad_ref, mx_ref, un0_ref, un1_ref, dc_ref,
              bias_ref, wroot_ref, x1_ref, root_ref):
    i = pl.program_id(0)
    shift = mx_ref[0, 0] + mx_ref[1, 0]
    z = as_ref[...] + ad_ref[...]
    ea = jnp.exp(jnp.maximum(z, 0.2 * z) - shift)
    den = dc_ref[:, 0:1] + dc_ref[:, 1:2] + ea
    un = un0_ref[...] + un1_ref[...] + ea * h_ref[...]
    x1 = jnp.maximum(un / den + bias_ref[...], 0.0)
    rowid = lax.broadcasted_iota(jnp.int32, (BLK, 1), 0) + i * BLK
    x1 = jnp.where(rowid < N, x1, 0.0)
    x1_ref[...] = x1
    root_ref[...] = jnp.dot(x1, wroot_ref[...], preferred_element_type=jnp.float32)


def _tc2(h, as_col, ad_col, mx, un0, un1, dcol, gat_bias_row, gc_Wroot):
    return pl.pallas_call(
        _tc2_body,
        grid=(GRID,),
        in_specs=[
            pl.BlockSpec((BLK, D), lambda i: (i, 0)),
            pl.BlockSpec((BLK, 1), lambda i: (i, 0)),
            pl.BlockSpec((BLK, 1), lambda i: (i, 0)),
            pl.BlockSpec((2, D), lambda i: (0, 0)),
            pl.BlockSpec((BLK, D), lambda i: (i, 0)),
            pl.BlockSpec((BLK, D), lambda i: (i, 0)),
            pl.BlockSpec((BLK, 2), lambda i: (i, 0)),
            pl.BlockSpec((1, D), lambda i: (0, 0)),
            pl.BlockSpec((D, D), lambda i: (0, 0)),
        ],
        out_specs=[
            pl.BlockSpec((BLK, D), lambda i: (i, 0)),
            pl.BlockSpec((BLK, D), lambda i: (i, 0)),
        ],
        out_shape=[
            jax.ShapeDtypeStruct((NPAD, D), jnp.float32),
            jax.ShapeDtypeStruct((NPAD, D), jnp.float32),
        ],
    )(h, as_col, ad_col, mx, un0, un1, dcol, gat_bias_row, gc_Wroot)


def _tc3_body(ag0_ref, ag1_ref, root_ref, wrel_ref, gcb_ref, w1_ref, b1_ref,
              w2_ref, b2_ref, y_ref):
    agg = ag0_ref[...] + ag1_ref[...]
    x2 = jnp.dot(agg, wrel_ref[...], preferred_element_type=jnp.float32)
    x2 = jnp.maximum(x2 + root_ref[...] + gcb_ref[...], 0.0)
    x3 = jnp.maximum(
        jnp.dot(x2, w1_ref[...], preferred_element_type=jnp.float32) + b1_ref[...],
        0.0)
    y_ref[...] = jnp.dot(x3, w2_ref[...], preferred_element_type=jnp.float32) + b2_ref[...]


def _tc3(ag0, ag1, root, gc_Wrel, gcb_row, lin1_W, b1_row, lin2_W, b2_row):
    return pl.pallas_call(
        _tc3_body,
        grid=(GRID,),
        in_specs=[
            pl.BlockSpec((BLK, D), lambda i: (i, 0)),
            pl.BlockSpec((BLK, D), lambda i: (i, 0)),
            pl.BlockSpec((BLK, D), lambda i: (i, 0)),
            pl.BlockSpec((D, D), lambda i: (0, 0)),
            pl.BlockSpec((1, D), lambda i: (0, 0)),
            pl.BlockSpec((D, D), lambda i: (0, 0)),
            pl.BlockSpec((1, D), lambda i: (0, 0)),
            pl.BlockSpec((D, D), lambda i: (0, 0)),
            pl.BlockSpec((1, D), lambda i: (0, 0)),
        ],
        out_specs=pl.BlockSpec((BLK, D), lambda i: (i, 0)),
        out_shape=jax.ShapeDtypeStruct((NPAD, D), jnp.float32),
    )(ag0, ag1, root, gc_Wrel, gcb_row, lin1_W, b1_row, lin2_W, b2_row)


def _sc_mesh():
    return plsc.VectorSubcoreMesh(core_axis_name="c", subcore_axis_name="s",
                                  num_cores=NC, num_subcores=NS)


def _sc_gat_body(src_hbm, dst_hbm, asrc_hbm, adst_hbm, h_hbm, bv_hbm,
                 un_out, den_out,
                 asrc_v, adst_v, bv_v, idx_s, idx_d, e_buf, rows, zbuf,
                 un_sp, den_sp, gsem):
    cid = lax.axis_index("c")
    sid = lax.axis_index("s")
    mych = jnp.where(cid == 0, C0, C1)
    base0 = jnp.where(cid == 0, sid * C0, NS * C0 + sid * C1) * CH

    def _zr(j, _):
        for cc in range(8):
            rows[j, pl.ds(cc * LANES, LANES)] = jnp.zeros((LANES,), jnp.float32)
        return 0
    lax.fori_loop(0, CH, _zr, 0)

    def _zz(j, _):
        zbuf[pl.ds(j * LANES, LANES)] = jnp.zeros((LANES,), jnp.float32)
        return 0
    lax.fori_loop(0, RPT // LANES, _zz, 0)

    pltpu.sync_copy(zbuf, den_sp.at[pl.ds(RPT * sid, RPT)])
    for k in range(RPT // CH):
        pltpu.sync_copy(rows, un_sp.at[pl.ds(RPT * sid + CH * k, CH)])
    plsc.subcore_barrier()

    pltpu.sync_copy(asrc_hbm, asrc_v)
    pltpu.sync_copy(adst_hbm, adst_v)
    pltpu.sync_copy(bv_hbm, bv_v)
    bv = bv_v[...]

    def chunk(c, _):
        base = pl.multiple_of(base0 + c * CH, CH)
        pltpu.sync_copy(src_hbm.at[pl.ds(base, CH)], idx_s)
        pltpu.sync_copy(dst_hbm.at[pl.ds(base, CH)], idx_d)
        gcopy = pltpu.async_copy(h_hbm.at[idx_s], rows, gsem)
        for j in range(CH // LANES):
            isv = idx_s[pl.ds(j * LANES, LANES)]
            idv = idx_d[pl.ds(j * LANES, LANES)]
            asv = plsc.load_gather(asrc_v, [isv])
            adv = plsc.load_gather(adst_v, [idv])
            z = asv + adv
            z = jnp.maximum(z, 0.2 * z) - bv
            e = jnp.exp(z)
            pos = lax.iota(jnp.int32, LANES) + (base + j * LANES)
            e = jnp.where(pos < E, e, 0.0)
            e_buf[pl.ds(j * LANES, LANES)] = e
        pltpu.sync_copy(e_buf, den_sp.at[idx_d], add=True)
        gcopy.wait()

        def scale(g, _):
            ev = e_buf[pl.ds(g * LANES, LANES)]
            for t in range(LANES):
                es = ev[t]
                j = g * LANES + t
                for cc in range(8):
                    rows[j, pl.ds(cc * LANES, LANES)] = rows[j, pl.ds(cc * LANES, LANES)] * es
            return 0
        lax.fori_loop(0, CH // LANES, scale, 0)
        pltpu.sync_copy(rows, un_sp.at[idx_d], add=True)
        return 0

    lax.fori_loop(0, mych, chunk, 0)
    plsc.subcore_barrier()

    pltpu.sync_copy(un_sp.at[pl.ds(RPT * sid, RPT)],
                    un_out.at[cid, pl.ds(RPT * sid, RPT)])
    pltpu.sync_copy(den_sp.at[pl.ds(RPT * sid, RPT)], den_out.at[cid, sid])


def _sc_gat(srcp, dstp, asrc, adst, h, bvec):
    return pl.kernel(
        _sc_gat_body,
        out_type=(
            jax.ShapeDtypeStruct((NC, NPAD, D), jnp.float32),
            jax.ShapeDtypeStruct((NC, NS, RPT), jnp.float32),
        ),
        mesh=_sc_mesh(),
        compiler_params=pltpu.CompilerParams(needs_layout_passes=False),
        scratch_types=[
            pltpu.VMEM((NPAD,), jnp.float32),
            pltpu.VMEM((NPAD,), jnp.float32),
            pltpu.VMEM((LANES,), jnp.float32),
            pltpu.VMEM((CH,), jnp.int32),
            pltpu.VMEM((CH,), jnp.int32),
            pltpu.VMEM((CH,), jnp.float32),
            pltpu.VMEM((CH, D), jnp.float32),
            pltpu.VMEM((RPT,), jnp.float32),
            pltpu.VMEM_SHARED((NPAD, D), jnp.float32),
            pltpu.VMEM_SHARED((NPAD,), jnp.float32),
            pltpu.SemaphoreType.DMA,
        ],
    )(srcp, dstp, asrc, adst, h, bvec)


def _sc_agg_body(src_hbm, dst_hbm, x1_hbm, ag_out,
                 idx_s, idx_d, rows, agg_sp, gsem):
    cid = lax.axis_index("c")
    sid = lax.axis_index("s")
    mych = jnp.where(cid == 0, C0, C1)
    base0 = jnp.where(cid == 0, sid * C0, NS * C0 + sid * C1) * CH

    def _zr(j, _):
        for cc in range(8):
            rows[j, pl.ds(cc * LANES, LANES)] = jnp.zeros((LANES,), jnp.float32)
        return 0
    lax.fori_loop(0, CH, _zr, 0)
    for k in range(RPT // CH):
        pltpu.sync_copy(rows, agg_sp.at[pl.ds(RPT * sid + CH * k, CH)])
    plsc.subcore_barrier()

    def chunk(c, _):
        base = pl.multiple_of(base0 + c * CH, CH)
        pltpu.sync_copy(src_hbm.at[pl.ds(base, CH)], idx_s)
        gcopy = pltpu.async_copy(x1_hbm.at[idx_s], rows, gsem)
        pltpu.sync_copy(dst_hbm.at[pl.ds(base, CH)], idx_d)
        gcopy.wait()
        pltpu.sync_copy(rows, agg_sp.at[idx_d], add=True)
        return 0

    lax.fori_loop(0, mych, chunk, 0)
    plsc.subcore_barrier()
    pltpu.sync_copy(agg_sp.at[pl.ds(RPT * sid, RPT)],
                    ag_out.at[cid, pl.ds(RPT * sid, RPT)])


def _sc_agg(srcp, dstp, x1):
    return pl.kernel(
        _sc_agg_body,
        out_type=jax.ShapeDtypeStruct((NC, NPAD, D), jnp.float32),
        mesh=_sc_mesh(),
        compiler_params=pltpu.CompilerParams(needs_layout_passes=False),
        scratch_types=[
            pltpu.VMEM((CH,), jnp.int32),
            pltpu.VMEM((CH,), jnp.int32),
            pltpu.VMEM((CH, D), jnp.float32),
            pltpu.VMEM_SHARED((NPAD, D), jnp.float32),
            pltpu.SemaphoreType.DMA,
        ],
    )(srcp, dstp, x1)


@jax.jit
def kernel(x, edge_index, edge_attr, batch, gat_W, gat_att_src, gat_att_dst,
           gat_bias, gc_Wrel, gc_Wroot, gc_bias, lin1_W, lin1_b, lin2_W,
           lin2_b):
    x_pad = jnp.zeros((NPAD, D), jnp.float32).at[:N].set(x)
    srcp = jnp.concatenate([
        edge_index[0].astype(jnp.int32),
        jnp.full((EPAD - E,), N, jnp.int32),
    ])
    dstp = jnp.concatenate([
        edge_index[1].astype(jnp.int32),
        jnp.zeros((EPAD - E,), jnp.int32),
    ])

    h, as_col, ad_col, mx = _tc1(x_pad, gat_W,
                                 gat_att_src.reshape(1, D),
                                 gat_att_dst.reshape(1, D))
    bvec = jnp.full((LANES,), mx[0, 0] + mx[1, 0], jnp.float32)

    un_p, den_p = _sc_gat(srcp, dstp, as_col.reshape(NPAD), ad_col.reshape(NPAD),
                          h, bvec)
    dcol = den_p.reshape(NC, NPAD).T

    x1, root = _tc2(h, as_col, ad_col, mx, un_p[0], un_p[1], dcol,
                    gat_bias.reshape(1, D), gc_Wroot)

    ag_p = _sc_agg(srcp, dstp, x1)

    y = _tc3(ag_p[0], ag_p[1], root, gc_Wrel, gc_bias.reshape(1, D),
             lin1_W, lin1_b.reshape(1, D), lin2_W, lin2_b.reshape(1, D))
    return y[:N]
